# baseline (speedup 1.0000x reference)
import functools

import jax
import jax.numpy as jnp
from jax import lax
from jax.experimental import pallas as pl
from jax.experimental.pallas import tpu as pltpu
from jax.experimental.pallas import tpu_sc as plsc

VOCAB = 1000000
DIM = 32
BATCH = 16384
NEG = 20

NW = 32
B_PER_W = BATCH // NW
CHUNK = 64
NCHUNK = B_PER_W // CHUNK
NEG_ROWS = CHUNK * NEG
NEG_GATHERS = NEG_ROWS // 128
QUAD = 4 * CHUNK

RB = 2048
SLAB = 123 * RB
V32 = 4 * SLAB


def _repack(table_t):
    def body(a_ref, b_ref, c_ref, d_ref, o_ref):
        o_ref[...] = jnp.concatenate(
            [
                jnp.transpose(a_ref[...]),
                jnp.transpose(b_ref[...]),
                jnp.transpose(c_ref[...]),
                jnp.transpose(d_ref[...]),
            ],
            axis=1,
        )

    nb = SLAB // RB
    last = VOCAB // RB
    in_specs = [
        pl.BlockSpec((32, RB), (lambda r, s=s: (0, jnp.minimum(s * nb + r, last))))
        for s in range(4)
    ]
    return pl.pallas_call(
        body,
        grid=(nb,),
        in_specs=in_specs,
        out_specs=pl.BlockSpec((RB, 128), lambda r: (r, 0)),
        out_shape=jax.ShapeDtypeStruct((SLAB, 128), jnp.float32),
    )(table_t, table_t, table_t, table_t)


def _sc_scores(in_lin, out_v, crow, xrow, nrow):
    mesh = plsc.VectorSubcoreMesh(core_axis_name="c", subcore_axis_name="s")

    @functools.partial(
        pl.kernel,
        mesh=mesh,
        compiler_params=pltpu.CompilerParams(
            needs_layout_passes=False, use_tc_tiling_on_sc=False
        ),
        out_type=jax.ShapeDtypeStruct((NW, 1 + NEG, B_PER_W), jnp.float32),
        scratch_types=[
            pltpu.VMEM((NCHUNK, CHUNK), jnp.int32),
            pltpu.VMEM((NCHUNK, CHUNK), jnp.int32),
            pltpu.VMEM((NCHUNK, NEG_GATHERS, 128), jnp.int32),
            pltpu.VMEM((CHUNK, DIM), jnp.float32),
            pltpu.VMEM((CHUNK, DIM), jnp.float32),
            pltpu.VMEM((CHUNK, DIM), jnp.float32),
            pltpu.VMEM((CHUNK, DIM), jnp.float32),
            pltpu.VMEM((NEG_ROWS, DIM), jnp.float32),
            pltpu.VMEM((NEG_ROWS, DIM), jnp.float32),
            pltpu.VMEM((1 + NEG, QUAD), jnp.float32),
            pltpu.SemaphoreType.DMA,
            pltpu.SemaphoreType.DMA,
        ],
    )
    def k(in_h, out_h, crow_h, xrow_h, nrow_h, scores_h,
          crow_v, xrow_v, nrow_v, ctr_a, ctr_b, ctx_a, ctx_b, neg_a, neg_b,
          sc_v, sem_a, sem_b):
        wid = lax.axis_index("s") * 2 + lax.axis_index("c")
        pltpu.sync_copy(crow_h.at[wid], crow_v)
        pltpu.sync_copy(xrow_h.at[wid], xrow_v)
        pltpu.sync_copy(nrow_h.at[wid], nrow_v)

        iota = lax.iota(jnp.int32, 16)
        bufs = ((ctr_a, ctx_a, neg_a, sem_a), (ctr_b, ctx_b, neg_b, sem_b))

        def fire(c, buf):
            ctr_v, ctx_v, neg_v, sem = buf
            pltpu.async_copy(in_h.at[crow_v.at[c]], ctr_v, sem)
            pltpu.async_copy(out_h.at[xrow_v.at[c]], ctx_v, sem)
            for j in range(NEG_GATHERS):
                pltpu.async_copy(
                    out_h.at[nrow_v.at[c, j]],
                    neg_v.at[pl.ds(j * 128, 128)],
                    sem,
                )

        def drain(c, buf):
            ctr_v, ctx_v, neg_v, sem = buf
            pltpu.make_async_copy(in_h.at[crow_v.at[c]], ctr_v, sem).wait()
            pltpu.make_async_copy(out_h.at[xrow_v.at[c]], ctx_v, sem).wait()
            for j in range(NEG_GATHERS):
                pltpu.make_async_copy(
                    out_h.at[nrow_v.at[c, j]],
                    neg_v.at[pl.ds(j * 128, 128)],
                    sem,
                ).wait()

        def compute(c, buf):
            ctr_v, ctx_v, neg_v, _ = buf
            qb = (c % 4) * CHUNK

            def group_body(g, _):
                rb = g * 16
                ob = qb + rb
                row16 = rb + iota
                nbase = row16 * NEG
                acc_p = jnp.zeros((16,), jnp.float32)
                acc_n = [jnp.zeros((16,), jnp.float32) for _ in range(NEG)]
                for d in range(DIM):
                    colv = jnp.full((16,), d, jnp.int32)
                    cd = plsc.load_gather(ctr_v, [row16, colv])
                    xd = plsc.load_gather(ctx_v, [row16, colv])
                    acc_p = acc_p + cd * xd
                    for kk in range(NEG):
                        nd = plsc.load_gather(neg_v, [nbase + kk, colv])
                        acc_n[kk] = acc_n[kk] - cd * nd
                sc_v[0, pl.ds(ob, 16)] = acc_p
                for kk in range(NEG):
                    sc_v[1 + kk, pl.ds(ob, 16)] = acc_n[kk]
                return ()

            lax.fori_loop(0, CHUNK // 16, group_body, ())

        fire(0, bufs[0])

        def pair_body(i, _):
            c0 = 2 * i
            c1 = c0 + 1
            fire(c1, bufs[1])
            drain(c0, bufs[0])
            compute(c0, bufs[0])

            @pl.when(i < NCHUNK // 2 - 1)
            def _():
                fire(c0 + 2, bufs[0])

            drain(c1, bufs[1])
            compute(c1, bufs[1])

            @pl.when(i % 2 == 1)
            def _():
                pltpu.sync_copy(
                    sc_v,
                    scores_h.at[wid, :, pl.ds((i // 2) * QUAD, QUAD)],
                )

            return ()

        lax.fori_loop(0, NCHUNK // 2, pair_body, ())

    return k(in_lin, out_v, crow, xrow, nrow)


def _tc_loss(scores):
    def body(s_ref, o_ref):
        x = s_ref[...]
        sig = 1.0 / (1.0 + jnp.exp(-x))
        o_ref[0, 0] = -jnp.sum(jnp.log(sig + 1e-9)) / BATCH

    return pl.pallas_call(
        body,
        out_shape=jax.ShapeDtypeStruct((1, 1), jnp.float32),
        out_specs=pl.BlockSpec(memory_space=pltpu.SMEM),
    )(scores)


def kernel(input_embeddings, output_embeddings, center_words, context_words,
           negative_words):
    out_v = _repack(output_embeddings.T).reshape(V32, DIM)

    def packed_row(w):
        return (w % SLAB) * 4 + w // SLAB

    cw = center_words.astype(jnp.int32)
    xw = packed_row(context_words.astype(jnp.int32))
    nw = packed_row(negative_words.astype(jnp.int32))
    crow = cw.reshape(NW, NCHUNK, CHUNK)
    xrow = xw.reshape(NW, NCHUNK, CHUNK)
    nrow = nw.reshape(NW, NCHUNK, NEG_GATHERS, 128)
    scores = _sc_scores(input_embeddings, out_v, crow, xrow, nrow)
    loss = _tc_loss(scores)
    return loss[0, 0]


# speedup vs baseline: 1.1491x; 1.1491x over previous
import functools

import jax
import jax.numpy as jnp
from jax import lax
from jax.experimental import pallas as pl
from jax.experimental.pallas import tpu as pltpu
from jax.experimental.pallas import tpu_sc as plsc

VOCAB = 1000000
DIM = 32
BATCH = 16384
NEG = 20

NW = 32
B_PER_W = BATCH // NW
CHUNK = 64
NCHUNK = B_PER_W // CHUNK
NEG_ROWS = CHUNK * NEG
NEG_GATHERS = NEG_ROWS // 128
QUAD = 4 * CHUNK

RB = 4096
SLAB = 62 * RB
V32 = 4 * SLAB


def _repack(table_t):
    def body(a_ref, b_ref, c_ref, d_ref, o_ref):
        o_ref[...] = jnp.concatenate(
            [
                jnp.transpose(a_ref[...]),
                jnp.transpose(b_ref[...]),
                jnp.transpose(c_ref[...]),
                jnp.transpose(d_ref[...]),
            ],
            axis=1,
        )

    nb = SLAB // RB
    last = VOCAB // RB
    in_specs = [
        pl.BlockSpec((32, RB), (lambda r, s=s: (0, jnp.minimum(s * nb + r, last))))
        for s in range(4)
    ]
    return pl.pallas_call(
        body,
        grid=(nb,),
        in_specs=in_specs,
        out_specs=pl.BlockSpec((RB, 128), lambda r: (r, 0)),
        out_shape=jax.ShapeDtypeStruct((SLAB, 128), jnp.float32),
    )(table_t, table_t, table_t, table_t)


def _sc_scores(in_lin, out_v, crow, xrow, nrow):
    mesh = plsc.VectorSubcoreMesh(core_axis_name="c", subcore_axis_name="s")

    @functools.partial(
        pl.kernel,
        mesh=mesh,
        compiler_params=pltpu.CompilerParams(
            needs_layout_passes=False, use_tc_tiling_on_sc=False
        ),
        out_type=jax.ShapeDtypeStruct((NW, 1 + NEG, B_PER_W), jnp.float32),
        scratch_types=[
            pltpu.VMEM((NCHUNK, CHUNK), jnp.int32),
            pltpu.VMEM((NCHUNK, CHUNK), jnp.int32),
            pltpu.VMEM((NCHUNK, NEG_GATHERS, 128), jnp.int32),
            pltpu.VMEM((CHUNK, DIM), jnp.float32),
            pltpu.VMEM((CHUNK, DIM), jnp.float32),
            pltpu.VMEM((CHUNK, DIM), jnp.float32),
            pltpu.VMEM((CHUNK, DIM), jnp.float32),
            pltpu.VMEM((NEG_ROWS, DIM), jnp.float32),
            pltpu.VMEM((NEG_ROWS, DIM), jnp.float32),
            pltpu.VMEM((1 + NEG, QUAD), jnp.float32),
            pltpu.SemaphoreType.DMA,
            pltpu.SemaphoreType.DMA,
        ],
    )
    def k(in_h, out_h, crow_h, xrow_h, nrow_h, scores_h,
          crow_v, xrow_v, nrow_v, ctr_a, ctr_b, ctx_a, ctx_b, neg_a, neg_b,
          sc_v, sem_a, sem_b):
        wid = lax.axis_index("s") * 2 + lax.axis_index("c")
        pltpu.sync_copy(crow_h.at[wid], crow_v)
        pltpu.sync_copy(xrow_h.at[wid], xrow_v)
        pltpu.sync_copy(nrow_h.at[wid], nrow_v)

        iota = lax.iota(jnp.int32, 16)
        bufs = ((ctr_a, ctx_a, neg_a, sem_a), (ctr_b, ctx_b, neg_b, sem_b))

        def fire(c, buf):
            ctr_v, ctx_v, neg_v, sem = buf
            pltpu.async_copy(in_h.at[crow_v.at[c]], ctr_v, sem)
            pltpu.async_copy(out_h.at[xrow_v.at[c]], ctx_v, sem)
            for j in range(NEG_GATHERS):
                pltpu.async_copy(
                    out_h.at[nrow_v.at[c, j]],
                    neg_v.at[pl.ds(j * 128, 128)],
                    sem,
                )

        def drain(c, buf):
            ctr_v, ctx_v, neg_v, sem = buf
            pltpu.make_async_copy(in_h.at[crow_v.at[c]], ctr_v, sem).wait()
            pltpu.make_async_copy(out_h.at[xrow_v.at[c]], ctx_v, sem).wait()
            for j in range(NEG_GATHERS):
                pltpu.make_async_copy(
                    out_h.at[nrow_v.at[c, j]],
                    neg_v.at[pl.ds(j * 128, 128)],
                    sem,
                ).wait()

        def compute(c, buf):
            ctr_v, ctx_v, neg_v, _ = buf
            qb = (c % 4) * CHUNK

            def group_body(g, _):
                rb = g * 16
                ob = qb + rb
                row16 = rb + iota
                nbase = row16 * NEG
                acc_p = jnp.zeros((16,), jnp.float32)
                acc_n = [jnp.zeros((16,), jnp.float32) for _ in range(NEG)]
                for d in range(DIM):
                    colv = jnp.full((16,), d, jnp.int32)
                    cd = plsc.load_gather(ctr_v, [row16, colv])
                    xd = plsc.load_gather(ctx_v, [row16, colv])
                    acc_p = acc_p + cd * xd
                    for kk in range(NEG):
                        nd = plsc.load_gather(neg_v, [nbase + kk, colv])
                        acc_n[kk] = acc_n[kk] - cd * nd
                sc_v[0, pl.ds(ob, 16)] = acc_p
                for kk in range(NEG):
                    sc_v[1 + kk, pl.ds(ob, 16)] = acc_n[kk]
                return ()

            lax.fori_loop(0, CHUNK // 16, group_body, ())

        fire(0, bufs[0])

        def pair_body(i, _):
            c0 = 2 * i
            c1 = c0 + 1
            fire(c1, bufs[1])
            drain(c0, bufs[0])
            compute(c0, bufs[0])

            @pl.when(i < NCHUNK // 2 - 1)
            def _():
                fire(c0 + 2, bufs[0])

            drain(c1, bufs[1])
            compute(c1, bufs[1])

            @pl.when(i % 2 == 1)
            def _():
                pltpu.sync_copy(
                    sc_v,
                    scores_h.at[wid, :, pl.ds((i // 2) * QUAD, QUAD)],
                )

            return ()

        lax.fori_loop(0, NCHUNK // 2, pair_body, ())

    return k(in_lin, out_v, crow, xrow, nrow)


def _tc_loss(scores):
    def body(s_ref, o_ref):
        x = s_ref[...]
        sig = 1.0 / (1.0 + jnp.exp(-x))
        o_ref[0, 0] = -jnp.sum(jnp.log(sig + 1e-9)) / BATCH

    return pl.pallas_call(
        body,
        out_shape=jax.ShapeDtypeStruct((1, 1), jnp.float32),
        out_specs=pl.BlockSpec(memory_space=pltpu.SMEM),
    )(scores)


def kernel(input_embeddings, output_embeddings, center_words, context_words,
           negative_words):
    in_v = _repack(input_embeddings.T).reshape(V32, DIM)
    out_v = _repack(output_embeddings.T).reshape(V32, DIM)

    def packed_row(w):
        return (w % SLAB) * 4 + w // SLAB

    cw = packed_row(center_words.astype(jnp.int32))
    xw = packed_row(context_words.astype(jnp.int32))
    nw = packed_row(negative_words.astype(jnp.int32))
    crow = cw.reshape(NW, NCHUNK, CHUNK)
    xrow = xw.reshape(NW, NCHUNK, CHUNK)
    nrow = nw.reshape(NW, NCHUNK, NEG_GATHERS, 128)
    scores = _sc_scores(in_v, out_v, crow, xrow, nrow)
    loss = _tc_loss(scores)
    return loss[0, 0]
